# baseline (speedup 1.0000x reference)
import functools

import jax
import jax.numpy as jnp
from jax import lax
from jax.experimental import pallas as pl
from jax.experimental.pallas import tpu as pltpu
from jax.experimental.pallas import tpu_sc as plsc

F = 26
K = 64
B = 16384
FIELD_SIZE = 100000
NC = 2
NS = 16
NW = NC * NS
BPW = B // NW
IDX_CHUNK = 128
NCHUNK = BPW // IDX_CHUNK


def _lane_shuffle(v, idx):
    return lax.gather(
        v,
        idx[:, None],
        dimension_numbers=lax.GatherDimensionNumbers(
            offset_dims=(), collapsed_slice_dims=(0,), start_index_map=(0,)
        ),
        slice_sizes=(1,),
        mode=lax.GatherScatterMode.PROMISE_IN_BOUNDS,
    )


def _sc_body(xt_hbm, table_hbm, out_hbm, idx_v, acc_v, sem):
    wid = lax.axis_index("s") * NC + lax.axis_index("c")
    base = wid * BPW

    pltpu.sync_copy(xt_hbm.at[:, pl.ds(base, BPW)], idx_v)

    def add_off(i, _):
        f = i // (BPW // 16)
        j = i % (BPW // 16)
        sl = pl.ds(j * 16, 16)
        idx_v[f, sl] = idx_v[f, sl] + f * FIELD_SIZE
        return 0

    lax.fori_loop(0, F * (BPW // 16), add_off, 0)

    zeros = jnp.zeros((16,), jnp.float32)

    def zero_body(i, _):
        acc_v[i // (K // 16), pl.ds((i % (K // 16)) * 16, 16)] = zeros
        return 0

    lax.fori_loop(0, BPW * K // 16, zero_body, 0)

    def fire(f, _):
        for c in range(NCHUNK):
            pltpu.async_copy(
                table_hbm.at[idx_v.at[f, pl.ds(c * IDX_CHUNK, IDX_CHUNK)]],
                acc_v.at[pl.ds(c * IDX_CHUNK, IDX_CHUNK)],
                sem,
                add=True,
            )
        return 0

    lax.fori_loop(0, F, fire, 0)

    def drain(i, _):
        pltpu.make_async_copy(
            table_hbm.at[pl.ds(0, IDX_CHUNK)],
            acc_v.at[pl.ds(0, IDX_CHUNK)],
            sem,
        ).wait()
        return 0

    lax.fori_loop(0, F * NCHUNK, drain, 0)

    lanes = lax.iota(jnp.int32, 16)

    def lane_sum(v):
        for sh in (8, 4, 2, 1):
            v = v + _lane_shuffle(v, lanes ^ sh)
        return v

    def softmax_row(r, _):
        e0 = jnp.exp(acc_v[r, pl.ds(0, 16)])
        e1 = jnp.exp(acc_v[r, pl.ds(16, 16)])
        e2 = jnp.exp(acc_v[r, pl.ds(32, 16)])
        e3 = jnp.exp(acc_v[r, pl.ds(48, 16)])
        inv = 1.0 / lane_sum((e0 + e1) + (e2 + e3))
        acc_v[r, pl.ds(0, 16)] = e0 * inv
        acc_v[r, pl.ds(16, 16)] = e1 * inv
        acc_v[r, pl.ds(32, 16)] = e2 * inv
        acc_v[r, pl.ds(48, 16)] = e3 * inv
        return 0

    lax.fori_loop(0, BPW, softmax_row, 0)

    pltpu.sync_copy(acc_v, out_hbm.at[pl.ds(base, BPW)])


_sc_call = functools.partial(
    pl.kernel,
    out_type=jax.ShapeDtypeStruct((B, K), jnp.float32),
    mesh=plsc.VectorSubcoreMesh(core_axis_name="c", subcore_axis_name="s"),
    scratch_types=[
        pltpu.VMEM((F, BPW), jnp.int32),
        pltpu.VMEM((BPW, K), jnp.float32),
        pltpu.SemaphoreType.DMA,
    ],
    compiler_params=pltpu.CompilerParams(use_tc_tiling_on_sc=False),
)(_sc_body)


@jax.jit
def kernel(x, table):
    xt = x.T
    return _sc_call(xt, table)


# speedup vs baseline: 1.0011x; 1.0011x over previous
import functools

import jax
import jax.numpy as jnp
from jax import lax
from jax.experimental import pallas as pl
from jax.experimental.pallas import tpu as pltpu
from jax.experimental.pallas import tpu_sc as plsc

F = 26
K = 64
B = 16384
FIELD_SIZE = 100000
NC = 2
NS = 16
NW = NC * NS
BPW = B // NW
IDX_CHUNK = 128
NCHUNK = BPW // IDX_CHUNK

BT = 2048


def _tc_transpose_body(x_ref, xt_ref):
    offs = lax.broadcasted_iota(jnp.int32, (F, BT), 0) * FIELD_SIZE
    xt_ref[...] = x_ref[...].T + offs


_tc_transpose = pl.pallas_call(
    _tc_transpose_body,
    grid=(B // BT,),
    in_specs=[pl.BlockSpec((BT, F), lambda i: (i, 0))],
    out_specs=pl.BlockSpec((F, BT), lambda i: (0, i)),
    out_shape=jax.ShapeDtypeStruct((F, B), jnp.int32),
)


def _lane_shuffle(v, idx):
    return lax.gather(
        v,
        idx[:, None],
        dimension_numbers=lax.GatherDimensionNumbers(
            offset_dims=(), collapsed_slice_dims=(0,), start_index_map=(0,)
        ),
        slice_sizes=(1,),
        mode=lax.GatherScatterMode.PROMISE_IN_BOUNDS,
    )


def _sc_body(xt_hbm, table_hbm, out_hbm, idx_v, acc_v, sem):
    wid = lax.axis_index("s") * NC + lax.axis_index("c")
    base = wid * BPW

    idx_cp = pltpu.async_copy(xt_hbm.at[:, pl.ds(base, BPW)], idx_v, sem)

    zeros = jnp.zeros((16,), jnp.float32)

    def zero_body(i, _):
        acc_v[i // (K // 16), pl.ds((i % (K // 16)) * 16, 16)] = zeros
        return 0

    lax.fori_loop(0, BPW * K // 16, zero_body, 0)
    idx_cp.wait()

    def fire(f, _):
        for c in range(NCHUNK):
            pltpu.async_copy(
                table_hbm.at[idx_v.at[f, pl.ds(c * IDX_CHUNK, IDX_CHUNK)]],
                acc_v.at[pl.ds(c * IDX_CHUNK, IDX_CHUNK)],
                sem,
                add=True,
            )
        return 0

    lax.fori_loop(0, F, fire, 0)

    def drain(i, _):
        pltpu.make_async_copy(
            table_hbm.at[pl.ds(0, IDX_CHUNK)],
            acc_v.at[pl.ds(0, IDX_CHUNK)],
            sem,
        ).wait()
        return 0

    lax.fori_loop(0, F * NCHUNK, drain, 0)

    lanes = lax.iota(jnp.int32, 16)

    def lane_sum(v):
        for sh in (8, 4, 2, 1):
            v = v + _lane_shuffle(v, lanes ^ sh)
        return v

    def softmax_row(r, _):
        e0 = jnp.exp(acc_v[r, pl.ds(0, 16)])
        e1 = jnp.exp(acc_v[r, pl.ds(16, 16)])
        e2 = jnp.exp(acc_v[r, pl.ds(32, 16)])
        e3 = jnp.exp(acc_v[r, pl.ds(48, 16)])
        inv = 1.0 / lane_sum((e0 + e1) + (e2 + e3))
        acc_v[r, pl.ds(0, 16)] = e0 * inv
        acc_v[r, pl.ds(16, 16)] = e1 * inv
        acc_v[r, pl.ds(32, 16)] = e2 * inv
        acc_v[r, pl.ds(48, 16)] = e3 * inv
        return 0

    lax.fori_loop(0, BPW, softmax_row, 0)

    pltpu.sync_copy(acc_v, out_hbm.at[pl.ds(base, BPW)])


_sc_call = functools.partial(
    pl.kernel,
    out_type=jax.ShapeDtypeStruct((B, K), jnp.float32),
    mesh=plsc.VectorSubcoreMesh(core_axis_name="c", subcore_axis_name="s"),
    scratch_types=[
        pltpu.VMEM((F, BPW), jnp.int32),
        pltpu.VMEM((BPW, K), jnp.float32),
        pltpu.SemaphoreType.DMA,
    ],
    compiler_params=pltpu.CompilerParams(use_tc_tiling_on_sc=False),
)(_sc_body)


@jax.jit
def kernel(x, table):
    xt = _tc_transpose(x)
    return _sc_call(xt, table)
